# baseline (speedup 1.0000x reference)
import jax
import jax.numpy as jnp
from jax import lax
from jax.experimental import pallas as pl
from jax.experimental.pallas import tpu as pltpu
from jax.experimental.pallas import tpu_sc as plsc

_NC, _NS = 2, 16
_NW = _NC * _NS
_G = 128
_GPC = 2
_NB = 3


def _gather_sc(table, slic, num_rows):
    B, S, dim = table.shape
    per_w = num_rows // _NW
    CHUNK = _GPC * _G
    K = per_w // CHUNK
    G_GROUPS = K // _NB
    w_per_b = _NW // B
    mesh = plsc.VectorSubcoreMesh(core_axis_name="core", subcore_axis_name="subcore")

    @pl.kernel(
        out_type=jax.ShapeDtypeStruct((num_rows, dim), jnp.float32),
        mesh=mesh,
        scratch_types=[
            pltpu.VMEM((K * _GPC, _G), jnp.int32),
            pltpu.VMEM((_NB, CHUNK, dim), jnp.float32),
            pltpu.VMEM((1, dim), jnp.float32),
            *[pltpu.VMEM_SHARED((S + 1, dim), jnp.float32) for _ in range(B)],
            pltpu.SemaphoreType.DMA,
            pltpu.SemaphoreType.DMA,
            pltpu.SemaphoreType.DMA,
        ],
    )
    def k(x_hbm, i_hbm, o_hbm, idx_v, rows_v, zrow_v, *tabs_and_sems):
        tabs, (gsem, wsem, tsem) = tabs_and_sems[:B], tabs_and_sems[B:]
        sid = lax.axis_index("subcore")
        wid = sid * _NC + lax.axis_index("core")
        base = wid * per_w
        batch = wid // w_per_b

        pltpu.async_copy(i_hbm.at[wid], idx_v, gsem)

        for b in range(B):

            @pl.when(sid == b)
            def _(b=b):
                for c in range(dim // 16):
                    zrow_v[0, pl.ds(c * 16, 16)] = jnp.zeros((16,), jnp.float32)
                pltpu.async_copy(x_hbm.at[b], tabs[b].at[pl.ds(1, S)], tsem)
                pltpu.async_copy(zrow_v, tabs[b].at[pl.ds(0, 1)], tsem)
                pltpu.make_async_copy(x_hbm.at[b], tabs[b].at[pl.ds(1, S)], tsem).wait()
                pltpu.make_async_copy(zrow_v, tabs[b].at[pl.ds(0, 1)], tsem).wait()

        pltpu.make_async_copy(i_hbm.at[wid], idx_v, gsem).wait()
        plsc.subcore_barrier()

        def ring(tab):
            def gathers(c, j):
                return [
                    pltpu.make_async_copy(
                        tab.at[idx_v.at[c * _GPC + p]],
                        rows_v.at[j, pl.ds(p * _G, _G)],
                        gsem,
                    )
                    for p in range(_GPC)
                ]

            def write(c, j):
                return pltpu.make_async_copy(
                    rows_v.at[j], o_hbm.at[pl.ds(base + c * CHUNK, CHUNK)], wsem
                )

            for j in range(_NB):
                for cp in gathers(j, j):
                    cp.start()

            @pl.loop(0, G_GROUPS - 1)
            def _(g):
                c0 = g * _NB
                for j in range(_NB):
                    for cp in gathers(c0 + j, j):
                        cp.wait()
                    write(c0 + j, j).start()
                for j in range(_NB):
                    write(c0 + j, j).wait()
                    for cp in gathers(c0 + _NB + j, j):
                        cp.start()

            c0 = (G_GROUPS - 1) * _NB
            for j in range(_NB):
                for cp in gathers(c0 + j, j):
                    cp.wait()
                write(c0 + j, j).start()
            for j in range(_NB):
                write(c0 + j, j).wait()

        for b in range(B):
            pl.when(batch == b)(lambda tab=tabs[b]: ring(tab))

    return k(table, slic)


def kernel(graph_lstm_output, slic_output):
    B, S, D = graph_lstm_output.shape
    _, H, W = slic_output.shape
    n = B * H * W
    slic = slic_output.reshape(_NW, n // (_NW * _G), _G)
    out = _gather_sc(graph_lstm_output, slic, n)
    return out.reshape(B, H, W, D)


# speedup vs baseline: 1.0083x; 1.0083x over previous
import jax
import jax.numpy as jnp
from jax import lax
from jax.experimental import pallas as pl
from jax.experimental.pallas import tpu as pltpu
from jax.experimental.pallas import tpu_sc as plsc

_NC, _NS = 2, 16
_NW = _NC * _NS
_G = 128
_GPC = 1
_NB = 4


def _gather_sc(table, slic, num_rows):
    B, S, dim = table.shape
    per_w = num_rows // _NW
    CHUNK = _GPC * _G
    K = per_w // CHUNK
    G_GROUPS = K // _NB
    w_per_b = _NW // B
    mesh = plsc.VectorSubcoreMesh(core_axis_name="core", subcore_axis_name="subcore")

    @pl.kernel(
        out_type=jax.ShapeDtypeStruct((num_rows, dim), jnp.float32),
        mesh=mesh,
        scratch_types=[
            pltpu.VMEM((K * _GPC, _G), jnp.int32),
            pltpu.VMEM((_NB, CHUNK, dim), jnp.float32),
            pltpu.VMEM((1, dim), jnp.float32),
            *[pltpu.VMEM_SHARED((S + 1, dim), jnp.float32) for _ in range(B)],
            pltpu.SemaphoreType.DMA,
            pltpu.SemaphoreType.DMA,
            pltpu.SemaphoreType.DMA,
        ],
    )
    def k(x_hbm, i_hbm, o_hbm, idx_v, rows_v, zrow_v, *tabs_and_sems):
        tabs, (gsem, wsem, tsem) = tabs_and_sems[:B], tabs_and_sems[B:]
        sid = lax.axis_index("subcore")
        wid = sid * _NC + lax.axis_index("core")
        base = wid * per_w
        batch = wid // w_per_b

        pltpu.async_copy(i_hbm.at[wid], idx_v, gsem)

        for b in range(B):

            @pl.when(sid == b)
            def _(b=b):
                for c in range(dim // 16):
                    zrow_v[0, pl.ds(c * 16, 16)] = jnp.zeros((16,), jnp.float32)
                pltpu.async_copy(x_hbm.at[b], tabs[b].at[pl.ds(1, S)], tsem)
                pltpu.async_copy(zrow_v, tabs[b].at[pl.ds(0, 1)], tsem)
                pltpu.make_async_copy(x_hbm.at[b], tabs[b].at[pl.ds(1, S)], tsem).wait()
                pltpu.make_async_copy(zrow_v, tabs[b].at[pl.ds(0, 1)], tsem).wait()

        pltpu.make_async_copy(i_hbm.at[wid], idx_v, gsem).wait()
        plsc.subcore_barrier()

        def ring(tab):
            def gathers(c, j):
                return [
                    pltpu.make_async_copy(
                        tab.at[idx_v.at[c * _GPC + p]],
                        rows_v.at[j, pl.ds(p * _G, _G)],
                        gsem,
                    )
                    for p in range(_GPC)
                ]

            def write(c, j):
                return pltpu.make_async_copy(
                    rows_v.at[j], o_hbm.at[pl.ds(base + c * CHUNK, CHUNK)], wsem
                )

            for j in range(_NB):
                for cp in gathers(j, j):
                    cp.start()

            @pl.loop(0, G_GROUPS - 1)
            def _(g):
                c0 = g * _NB
                for j in range(_NB):
                    for cp in gathers(c0 + j, j):
                        cp.wait()
                    write(c0 + j, j).start()
                for j in range(_NB):
                    write(c0 + j, j).wait()
                    for cp in gathers(c0 + _NB + j, j):
                        cp.start()

            c0 = (G_GROUPS - 1) * _NB
            for j in range(_NB):
                for cp in gathers(c0 + j, j):
                    cp.wait()
                write(c0 + j, j).start()
            for j in range(_NB):
                write(c0 + j, j).wait()

        for b in range(B):
            pl.when(batch == b)(lambda tab=tabs[b]: ring(tab))

    return k(table, slic)


def kernel(graph_lstm_output, slic_output):
    B, S, D = graph_lstm_output.shape
    _, H, W = slic_output.shape
    n = B * H * W
    slic = slic_output.reshape(_NW, n // (_NW * _G), _G)
    out = _gather_sc(graph_lstm_output, slic, n)
    return out.reshape(B, H, W, D)
